# baseline (speedup 1.0000x reference)
import functools

import jax
import jax.numpy as jnp
from jax import lax
from jax.experimental import pallas as pl
from jax.experimental.pallas import tpu as pltpu
from jax.experimental.pallas import tpu_sc as plsc

_LANES = 16
_NUM_TILES = 32


def _sc_body(n_jobs, rows_per_tile, chunk_rows, vocab, emb_dim,
             pt_hbm, asg_hbm, emb_hbm, w_hbm, b_hbm, out_hbm,
             emb_v, w_v, b_v, accbuf, etab, pt_buf, asg_buf, out_buf):
    tile = lax.axis_index("s") * 2 + lax.axis_index("c")
    iota = lax.iota(jnp.int32, _LANES)
    zeros_i = jnp.zeros((_LANES,), jnp.int32)

    pltpu.sync_copy(emb_hbm, emb_v)
    pltpu.sync_copy(w_hbm, w_v)
    pltpu.sync_copy(b_hbm, b_v)
    wvecs = [plsc.load_gather(w_v, [k * _LANES + iota, zeros_i])
             for k in range(emb_dim // _LANES)]
    for v in range(vocab):
        acc = jnp.zeros((_LANES,), jnp.float32)
        for k in range(emb_dim // _LANES):
            acc = acc + emb_v[v, pl.ds(k * _LANES, _LANES)] * wvecs[k]
        accbuf[pl.ds(v * _LANES, _LANES)] = acc
    tvec = jnp.zeros((_LANES,), jnp.float32)
    for l in range(_LANES):
        tvec = tvec + plsc.load_gather(accbuf, [iota * _LANES + l])
    bvec = plsc.load_gather(b_v, [zeros_i])
    tvec = jnp.where(iota < vocab, jnp.exp(tvec + bvec), 0.0)
    etab[...] = tvec

    n_chunks = rows_per_tile // chunk_rows
    n_full = n_jobs // _LANES
    tail0 = n_jobs - _LANES
    tail_new = n_jobs - n_full * _LANES
    row_base = tile * rows_per_tile
    last15 = jnp.full((_LANES,), _LANES - 1, jnp.int32)

    for chunk in range(n_chunks):
        r0 = row_base + chunk * chunk_rows
        pltpu.sync_copy(pt_hbm.at[pl.ds(r0, chunk_rows), :], pt_buf)
        pltpu.sync_copy(asg_hbm.at[pl.ds(r0, chunk_rows), :], asg_buf)

        @plsc.parallel_loop(0, chunk_rows, unroll=2)
        def row_body(r):
            evs = []
            acc = jnp.zeros((_LANES,), jnp.float32)
            for c in range(n_full):
                ptv = pt_buf[r, pl.ds(c * _LANES, _LANES)]
                av = asg_buf[r, pl.ds(c * _LANES, _LANES)]
                ev = plsc.load_gather(etab, [ptv])
                ev = jnp.where(av > 0, 0.0, ev)
                evs.append(ev)
                acc = acc + ev
            ptv = pt_buf[r, pl.ds(tail0, _LANES)]
            av = asg_buf[r, pl.ds(tail0, _LANES)]
            ev = plsc.load_gather(etab, [ptv])
            ev = jnp.where(av > 0, 0.0, ev)
            evs.append(ev)
            acc = acc + jnp.where(iota >= _LANES - tail_new, ev, 0.0)

            total = jnp.cumsum(acc).at[last15].get(mode="promise_in_bounds")
            recip = 1.0 / total
            for c in range(n_full):
                out_buf[r, pl.ds(c * _LANES, _LANES)] = evs[c] * recip
            out_buf[r, pl.ds(tail0, _LANES)] = evs[n_full] * recip

        pltpu.sync_copy(out_buf, out_hbm.at[pl.ds(r0, chunk_rows), :])


@functools.partial(jax.jit, static_argnames=("chunk_rows",))
def _sc_call(pt, asg, emb, w, b, *, chunk_rows=128):
    bsz, n_jobs = pt.shape
    vocab, emb_dim = emb.shape
    rows_per_tile = bsz // _NUM_TILES
    mesh = plsc.VectorSubcoreMesh(core_axis_name="c", subcore_axis_name="s")
    body = functools.partial(_sc_body, n_jobs, rows_per_tile, chunk_rows,
                             vocab, emb_dim)
    return pl.kernel(
        body,
        out_type=jax.ShapeDtypeStruct((bsz, n_jobs), jnp.float32),
        mesh=mesh,
        compiler_params=pltpu.CompilerParams(needs_layout_passes=False,
                                             use_tc_tiling_on_sc=True),
        scratch_types=[
            pltpu.VMEM((vocab, emb_dim), jnp.float32),
            pltpu.VMEM((emb_dim, 1), jnp.float32),
            pltpu.VMEM((1,), jnp.float32),
            pltpu.VMEM((_LANES * _LANES,), jnp.float32),
            pltpu.VMEM((_LANES,), jnp.float32),
            pltpu.VMEM((chunk_rows, n_jobs), jnp.int32),
            pltpu.VMEM((chunk_rows, n_jobs), jnp.int32),
            pltpu.VMEM((chunk_rows, n_jobs), jnp.float32),
        ],
    )(pt, asg, emb, w, b)


def kernel(proc_times, assigned, machine_times, job_embed, fc_w, fc_b):
    return _sc_call(proc_times, assigned, job_embed, fc_w, fc_b)


# speedup vs baseline: 1.6226x; 1.6226x over previous
import functools

import jax
import jax.numpy as jnp
from jax import lax
from jax.experimental import pallas as pl
from jax.experimental.pallas import tpu as pltpu
from jax.experimental.pallas import tpu_sc as plsc

_LANES = 16
_NUM_TILES = 32


def _sc_body(n_jobs, cols_per_tile, chunk_cols, vocab, emb_dim,
             pt_hbm, asg_hbm, emb_hbm, w_hbm, b_hbm, out_hbm,
             emb_v, w_v, b_v, accbuf, etab, pt_buf, asg_buf, e_buf, out_buf):
    tile = lax.axis_index("s") * 2 + lax.axis_index("c")
    iota = lax.iota(jnp.int32, _LANES)
    zeros_i = jnp.zeros((_LANES,), jnp.int32)

    pltpu.sync_copy(emb_hbm, emb_v)
    pltpu.sync_copy(w_hbm, w_v)
    pltpu.sync_copy(b_hbm, b_v)
    for v in range(vocab):
        acc = jnp.zeros((_LANES,), jnp.float32)
        for k in range(emb_dim // _LANES):
            acc = acc + (emb_v[v, pl.ds(k * _LANES, _LANES)]
                         * w_v[0, pl.ds(k * _LANES, _LANES)])
        accbuf[pl.ds(v * _LANES, _LANES)] = acc
    tvec = jnp.zeros((_LANES,), jnp.float32)
    for l in range(_LANES):
        tvec = tvec + plsc.load_gather(accbuf, [iota * _LANES + l])
    bvec = plsc.load_gather(b_v, [zeros_i])
    tvec = jnp.where(iota < vocab, jnp.exp(tvec + bvec), 0.0)
    etab[...] = tvec

    n_chunks = cols_per_tile // chunk_cols
    n_groups = chunk_cols // _LANES
    col_base = tile * cols_per_tile

    for chunk in range(n_chunks):
        c0 = col_base + chunk * chunk_cols
        pltpu.sync_copy(pt_hbm.at[:, pl.ds(c0, chunk_cols)], pt_buf)
        pltpu.sync_copy(asg_hbm.at[:, pl.ds(c0, chunk_cols)], asg_buf)

        for g in range(n_groups):
            g0 = g * _LANES

            @plsc.parallel_loop(0, n_jobs, unroll=8,
                                carry=jnp.zeros((_LANES,), jnp.float32))
            def pass1(j, acc, g0=g0):
                ptv = pt_buf[j, pl.ds(g0, _LANES)]
                av = asg_buf[j, pl.ds(g0, _LANES)]
                ev = plsc.load_gather(etab, [ptv])
                ev = jnp.where(av > 0, 0.0, ev)
                e_buf[j, pl.ds(g0, _LANES)] = ev
                return acc + ev

            recip = 1.0 / pass1

            @plsc.parallel_loop(0, n_jobs, unroll=8)
            def pass2(j, g0=g0, recip=recip):
                ev = e_buf[j, pl.ds(g0, _LANES)]
                out_buf[j, pl.ds(g0, _LANES)] = ev * recip

        pltpu.sync_copy(out_buf, out_hbm.at[:, pl.ds(c0, chunk_cols)])


@functools.partial(jax.jit, static_argnames=("chunk_cols",))
def _sc_call(pt_t, asg_t, emb, w_t, b, *, chunk_cols=128):
    n_jobs, bsz = pt_t.shape
    vocab, emb_dim = emb.shape
    cols_per_tile = bsz // _NUM_TILES
    mesh = plsc.VectorSubcoreMesh(core_axis_name="c", subcore_axis_name="s")
    body = functools.partial(_sc_body, n_jobs, cols_per_tile, chunk_cols,
                             vocab, emb_dim)
    return pl.kernel(
        body,
        out_type=jax.ShapeDtypeStruct((n_jobs, bsz), jnp.float32),
        mesh=mesh,
        compiler_params=pltpu.CompilerParams(needs_layout_passes=False,
                                             use_tc_tiling_on_sc=True),
        scratch_types=[
            pltpu.VMEM((vocab, emb_dim), jnp.float32),
            pltpu.VMEM((1, emb_dim), jnp.float32),
            pltpu.VMEM((1,), jnp.float32),
            pltpu.VMEM((_LANES * _LANES,), jnp.float32),
            pltpu.VMEM((_LANES,), jnp.float32),
            pltpu.VMEM((n_jobs, chunk_cols), jnp.int32),
            pltpu.VMEM((n_jobs, chunk_cols), jnp.int32),
            pltpu.VMEM((n_jobs, chunk_cols), jnp.float32),
            pltpu.VMEM((n_jobs, chunk_cols), jnp.float32),
        ],
    )(pt_t, asg_t, emb, w_t, b)


def kernel(proc_times, assigned, machine_times, job_embed, fc_w, fc_b):
    out_t = _sc_call(proc_times.T, assigned.T, job_embed, fc_w.T, fc_b)
    return out_t.T


# speedup vs baseline: 1.6319x; 1.0057x over previous
import functools

import jax
import jax.numpy as jnp
from jax import lax
from jax.experimental import pallas as pl
from jax.experimental.pallas import tpu as pltpu
from jax.experimental.pallas import tpu_sc as plsc

_LANES = 16
_NUM_TILES = 32


def _sc_body(n_jobs, cols_per_tile, chunk_cols, vocab, emb_dim,
             pt_hbm, asg_hbm, emb_hbm, w_hbm, b_hbm, out_hbm,
             w_v, b_v, accbuf, etab, io_bufs, e_buf, in_sems, out_sems):
    tile = lax.axis_index("s") * 2 + lax.axis_index("c")
    iota = lax.iota(jnp.int32, _LANES)
    zeros_i = jnp.zeros((_LANES,), jnp.int32)

    pltpu.sync_copy(emb_hbm, e_buf.at[pl.ds(0, vocab), :])
    pltpu.sync_copy(w_hbm, w_v)
    pltpu.sync_copy(b_hbm, b_v)
    for v in range(vocab):
        acc = jnp.zeros((_LANES,), jnp.float32)
        for k in range(emb_dim // _LANES):
            acc = acc + (e_buf[v, pl.ds(k * _LANES, _LANES)]
                         * w_v[0, pl.ds(k * _LANES, _LANES)])
        accbuf[pl.ds(v * _LANES, _LANES)] = acc
    tvec = jnp.zeros((_LANES,), jnp.float32)
    for l in range(_LANES):
        tvec = tvec + plsc.load_gather(accbuf, [iota * _LANES + l])
    bvec = plsc.load_gather(b_v, [zeros_i])
    tvec = jnp.where(iota < vocab, jnp.exp(tvec + bvec), 0.0)
    etab[...] = tvec

    n_chunks = cols_per_tile // chunk_cols
    n_groups = chunk_cols // _LANES
    col_base = tile * cols_per_tile

    def start_in(k):
        c0 = col_base + k * chunk_cols
        s = k % 2
        pltpu.async_copy(pt_hbm.at[:, pl.ds(c0, chunk_cols)],
                         io_bufs[2 * s], in_sems[2 * s])
        pltpu.async_copy(asg_hbm.at[:, pl.ds(c0, chunk_cols)],
                         io_bufs[2 * s + 1], in_sems[2 * s + 1])

    def wait_in(k):
        s = k % 2
        for i in (2 * s, 2 * s + 1):
            pltpu.make_async_copy(pt_hbm.at[:, pl.ds(0, chunk_cols)],
                                  io_bufs[i], in_sems[i]).wait()

    start_in(0)
    out_started = [None, None]
    for k in range(n_chunks):
        s = k % 2
        wait_in(k)
        if k + 1 < n_chunks:
            if out_started[1 - s] is not None:
                out_started[1 - s].wait()
                out_started[1 - s] = None
            start_in(k + 1)
        pt_buf, asg_buf = io_bufs[2 * s], io_bufs[2 * s + 1]

        for g in range(n_groups):
            g0 = g * _LANES

            @plsc.parallel_loop(0, n_jobs, unroll=8,
                                carry=jnp.zeros((_LANES,), jnp.float32))
            def pass1(j, acc, g0=g0, pt_buf=pt_buf, asg_buf=asg_buf):
                ptv = plsc.bitcast(pt_buf[j, pl.ds(g0, _LANES)], jnp.int32)
                av = plsc.bitcast(asg_buf[j, pl.ds(g0, _LANES)], jnp.int32)
                ev = plsc.load_gather(etab, [ptv])
                ev = jnp.where(av > 0, 0.0, ev)
                e_buf[j, pl.ds(g0, _LANES)] = ev
                return acc + ev

            recip = 1.0 / pass1

            @plsc.parallel_loop(0, n_jobs, unroll=8)
            def pass2(j, g0=g0, recip=recip, pt_buf=pt_buf):
                ev = e_buf[j, pl.ds(g0, _LANES)]
                pt_buf[j, pl.ds(g0, _LANES)] = ev * recip

        c0 = col_base + k * chunk_cols
        out_started[s] = pltpu.async_copy(
            pt_buf, out_hbm.at[:, pl.ds(c0, chunk_cols)], out_sems[s])
    for s in range(2):
        if out_started[s] is not None:
            out_started[s].wait()


@functools.partial(jax.jit, static_argnames=("chunk_cols",))
def _sc_call(pt_t, asg_t, emb, w_t, b, *, chunk_cols=128):
    n_jobs, bsz = pt_t.shape
    vocab, emb_dim = emb.shape
    cols_per_tile = bsz // _NUM_TILES
    mesh = plsc.VectorSubcoreMesh(core_axis_name="c", subcore_axis_name="s")
    body = functools.partial(_sc_body, n_jobs, cols_per_tile, chunk_cols,
                             vocab, emb_dim)
    return pl.kernel(
        body,
        out_type=jax.ShapeDtypeStruct((n_jobs, bsz), jnp.float32),
        mesh=mesh,
        compiler_params=pltpu.CompilerParams(needs_layout_passes=False,
                                             use_tc_tiling_on_sc=True),
        scratch_types=[
            pltpu.VMEM((1, emb_dim), jnp.float32),
            pltpu.VMEM((1,), jnp.float32),
            pltpu.VMEM((_LANES * _LANES,), jnp.float32),
            pltpu.VMEM((_LANES,), jnp.float32),
            [pltpu.VMEM((n_jobs, chunk_cols), jnp.float32) for _ in range(4)],
            pltpu.VMEM((n_jobs, chunk_cols), jnp.float32),
            [pltpu.SemaphoreType.DMA for _ in range(4)],
            [pltpu.SemaphoreType.DMA for _ in range(2)],
        ],
    )(pt_t, asg_t, emb, w_t, b)


def kernel(proc_times, assigned, machine_times, job_embed, fc_w, fc_b):
    pt_f = lax.bitcast_convert_type(proc_times, jnp.float32)
    asg_f = lax.bitcast_convert_type(assigned, jnp.float32)
    out_t = _sc_call(pt_f.T, asg_f.T, job_embed, fc_w.T, fc_b)
    return out_t.T
